# baseline (speedup 1.0000x reference)
import functools

import jax
import jax.numpy as jnp
from jax import lax
from jax.experimental import pallas as pl
from jax.experimental.pallas import tpu as pltpu
from jax.experimental.pallas import tpu_sc as plsc

N_NODES = 10000
N_EDGES = 320000
D = 128

NC = 2
NS = 16
NW = NC * NS

EDGE_BATCH = 128
K_UNROLL = 16
N_GROUPS = 5
BATCHES_PER_TILE = K_UNROLL * N_GROUPS
EDGES_PER_TILE = BATCHES_PER_TILE * EDGE_BATCH
E_PAD = EDGES_PER_TILE * NW
N_PAD = 10240
ROWS_PER_TILE = N_PAD // NS


def _sc_aggregate(x, row, col, zeros_blk):
    mesh = plsc.VectorSubcoreMesh(core_axis_name="c", subcore_axis_name="s")

    @functools.partial(
        pl.kernel,
        mesh=mesh,
        out_type=jax.ShapeDtypeStruct((NC, N_PAD, D), jnp.float32),
        scratch_types=[
            pltpu.VMEM((K_UNROLL, EDGE_BATCH), jnp.int32),
            pltpu.VMEM((K_UNROLL, EDGE_BATCH), jnp.int32),
            pltpu.VMEM((2, EDGE_BATCH, D), jnp.float32),
            pltpu.VMEM_SHARED((N_PAD, D), jnp.float32),
            pltpu.SemaphoreType.DMA,
            pltpu.SemaphoreType.DMA,
        ],
    )
    def k(x_hbm, row_hbm, col_hbm, zeros_hbm, out_hbm,
          row_v, col_v, rows_v, agg, sem0, sem1):
        c = lax.axis_index("c")
        s = lax.axis_index("s")
        wid = s * NC + c
        sems = (sem0, sem1)

        pltpu.sync_copy(zeros_hbm, agg.at[pl.ds(s * ROWS_PER_TILE, ROWS_PER_TILE)])
        plsc.subcore_barrier()

        def group(g, carry):
            pltpu.sync_copy(row_hbm.at[wid, pl.ds(g * K_UNROLL, K_UNROLL)], row_v)
            pltpu.sync_copy(col_hbm.at[wid, pl.ds(g * K_UNROLL, K_UNROLL)], col_v)
            pltpu.async_copy(x_hbm.at[row_v.at[0]], rows_v.at[0], sems[0])
            for j in range(K_UNROLL):
                cur = j % 2
                nxt = (j + 1) % 2
                if j + 1 < K_UNROLL:
                    pltpu.async_copy(
                        x_hbm.at[row_v.at[j + 1]], rows_v.at[nxt], sems[nxt])
                pltpu.make_async_copy(
                    x_hbm.at[row_v.at[j]], rows_v.at[cur], sems[cur]).wait()
                pltpu.sync_copy(rows_v.at[cur], agg.at[col_v.at[j]], add=True)
            return carry

        lax.fori_loop(0, N_GROUPS, group, 0)
        plsc.subcore_barrier()

        r0 = s * ROWS_PER_TILE
        pltpu.sync_copy(agg.at[pl.ds(r0, ROWS_PER_TILE)],
                        out_hbm.at[c, pl.ds(r0, ROWS_PER_TILE)])

    return k(x, row, col, zeros_blk)


def _mlp_body(x_ref, p0_ref, p1_ref, w1_ref, b1_ref, w2_ref, b2_ref, o_ref):
    h = x_ref[...] + p0_ref[...] + p1_ref[...]
    h1 = jnp.dot(h, w1_ref[...], preferred_element_type=jnp.float32) + b1_ref[...]
    h1 = jnp.maximum(h1, 0.0)
    o_ref[...] = jnp.dot(h1, w2_ref[...], preferred_element_type=jnp.float32) + b2_ref[...]


def _tc_mlp(x, p0, p1, w1t, b1, w2t, b2):
    block = 2000
    grid = (N_NODES // block,)
    row_spec = pl.BlockSpec((block, D), lambda i: (i, 0))
    full_spec = pl.BlockSpec((D, D), lambda i: (0, 0))
    bias_spec = pl.BlockSpec((1, D), lambda i: (0, 0))
    return pl.pallas_call(
        _mlp_body,
        grid=grid,
        in_specs=[row_spec, row_spec, row_spec, full_spec, bias_spec, full_spec, bias_spec],
        out_specs=row_spec,
        out_shape=jax.ShapeDtypeStruct((N_NODES, D), jnp.float32),
    )(x, p0, p1, w1t, b1, w2t, b2)


@jax.jit
def kernel(x, edge_index, W1, b1, W2, b2):
    pad = E_PAD - N_EDGES
    row = jnp.concatenate(
        [edge_index[0].astype(jnp.int32), jnp.zeros((pad,), jnp.int32)])
    col = jnp.concatenate(
        [edge_index[1].astype(jnp.int32), jnp.full((pad,), N_NODES, jnp.int32)])
    row = row.reshape(NW, BATCHES_PER_TILE, EDGE_BATCH)
    col = col.reshape(NW, BATCHES_PER_TILE, EDGE_BATCH)
    zeros_blk = jnp.zeros((ROWS_PER_TILE, D), jnp.float32)
    partials = _sc_aggregate(x, row, col, zeros_blk)
    return _tc_mlp(x, partials[0, :N_NODES], partials[1, :N_NODES],
                   W1.T, b1.reshape(1, D), W2.T, b2.reshape(1, D))


# speedup vs baseline: 1.0135x; 1.0135x over previous
import functools

import jax
import jax.numpy as jnp
from jax import lax
from jax.experimental import pallas as pl
from jax.experimental.pallas import tpu as pltpu
from jax.experimental.pallas import tpu_sc as plsc

N_NODES = 10000
N_EDGES = 320000
D = 128

NC = 2
NS = 16
NW = NC * NS

EDGE_BATCH = 128
K_UNROLL = 2
N_GROUPS = 40
BATCHES_PER_TILE = K_UNROLL * N_GROUPS
EDGES_PER_TILE = BATCHES_PER_TILE * EDGE_BATCH
E_PAD = EDGES_PER_TILE * NW
N_PAD = 10240
ROWS_PER_TILE = N_PAD // NS


def _sc_aggregate(x, row, col, zeros_blk):
    mesh = plsc.VectorSubcoreMesh(core_axis_name="c", subcore_axis_name="s")

    @functools.partial(
        pl.kernel,
        mesh=mesh,
        out_type=jax.ShapeDtypeStruct((NC, N_PAD, D), jnp.float32),
        scratch_types=[
            pltpu.VMEM((EDGE_BATCH,), jnp.int32),
            pltpu.VMEM((EDGE_BATCH,), jnp.int32),
            pltpu.VMEM((EDGE_BATCH,), jnp.int32),
            pltpu.VMEM((EDGE_BATCH,), jnp.int32),
            pltpu.VMEM((2, EDGE_BATCH, D), jnp.float32),
            pltpu.VMEM_SHARED((N_PAD, D), jnp.float32),
            pltpu.SemaphoreType.DMA,
            pltpu.SemaphoreType.DMA,
            pltpu.SemaphoreType.DMA,
            pltpu.SemaphoreType.DMA,
        ],
    )
    def k(x_hbm, row_hbm, col_hbm, zeros_hbm, out_hbm,
          row0, row1, col0, col1, rows_v, agg, si0, si1, sg0, sg1):
        c = lax.axis_index("c")
        s = lax.axis_index("s")
        wid = s * NC + c
        rows = (row0, row1)
        cols = (col0, col1)
        si = (si0, si1)
        sg = (sg0, sg1)

        pltpu.sync_copy(zeros_hbm, agg.at[pl.ds(s * ROWS_PER_TILE, ROWS_PER_TILE)])
        plsc.subcore_barrier()

        base = pl.multiple_of(wid * EDGES_PER_TILE, 8)

        def idx_off(b):
            return pl.multiple_of(base + b * EDGE_BATCH, 8)

        def start_idx(b, slot):
            off = idx_off(b)
            pltpu.async_copy(row_hbm.at[pl.ds(off, EDGE_BATCH)], rows[slot], si[slot])
            pltpu.async_copy(col_hbm.at[pl.ds(off, EDGE_BATCH)], cols[slot], si[slot])

        def wait_idx(b, slot):
            off = idx_off(b)
            pltpu.make_async_copy(
                row_hbm.at[pl.ds(off, EDGE_BATCH)], rows[slot], si[slot]).wait()
            pltpu.make_async_copy(
                col_hbm.at[pl.ds(off, EDGE_BATCH)], cols[slot], si[slot]).wait()

        off0 = idx_off(0)
        pltpu.sync_copy(row_hbm.at[pl.ds(off0, EDGE_BATCH)], row0)
        pltpu.sync_copy(col_hbm.at[pl.ds(off0, EDGE_BATCH)], col0)
        pltpu.async_copy(x_hbm.at[row0], rows_v.at[0], sg0)
        start_idx(1, 1)

        def group(g, carry):
            for j in range(K_UNROLL):
                b = g * K_UNROLL + j
                p = j % 2
                q = (j + 1) % 2

                @pl.when(b + 1 < BATCHES_PER_TILE)
                def _():
                    wait_idx(b + 1, q)
                    pltpu.async_copy(x_hbm.at[rows[q]], rows_v.at[q], sg[q])

                pltpu.make_async_copy(
                    x_hbm.at[rows[p]], rows_v.at[p], sg[p]).wait()
                pltpu.sync_copy(rows_v.at[p], agg.at[cols[p]], add=True)

                @pl.when(b + 2 < BATCHES_PER_TILE)
                def _():
                    start_idx(b + 2, p)
            return carry

        lax.fori_loop(0, N_GROUPS, group, 0)
        plsc.subcore_barrier()

        r0 = s * ROWS_PER_TILE
        pltpu.sync_copy(agg.at[pl.ds(r0, ROWS_PER_TILE)],
                        out_hbm.at[c, pl.ds(r0, ROWS_PER_TILE)])

    return k(x, row, col, zeros_blk)


def _mlp_body(x_ref, p0_ref, p1_ref, w1_ref, b1_ref, w2_ref, b2_ref, o_ref):
    h = x_ref[...] + p0_ref[...] + p1_ref[...]
    h1 = jnp.dot(h, w1_ref[...], preferred_element_type=jnp.float32) + b1_ref[...]
    h1 = jnp.maximum(h1, 0.0)
    o_ref[...] = jnp.dot(h1, w2_ref[...], preferred_element_type=jnp.float32) + b2_ref[...]


def _tc_mlp(x, p0, p1, w1t, b1, w2t, b2):
    block = 2000
    grid = (N_NODES // block,)
    row_spec = pl.BlockSpec((block, D), lambda i: (i, 0))
    full_spec = pl.BlockSpec((D, D), lambda i: (0, 0))
    bias_spec = pl.BlockSpec((1, D), lambda i: (0, 0))
    return pl.pallas_call(
        _mlp_body,
        grid=grid,
        in_specs=[row_spec, row_spec, row_spec, full_spec, bias_spec, full_spec, bias_spec],
        out_specs=row_spec,
        out_shape=jax.ShapeDtypeStruct((N_NODES, D), jnp.float32),
    )(x, p0, p1, w1t, b1, w2t, b2)


@jax.jit
def kernel(x, edge_index, W1, b1, W2, b2):
    pad = E_PAD - N_EDGES
    row = jnp.concatenate(
        [edge_index[0].astype(jnp.int32), jnp.zeros((pad,), jnp.int32)])
    col = jnp.concatenate(
        [edge_index[1].astype(jnp.int32), jnp.full((pad,), N_NODES, jnp.int32)])
    zeros_blk = jnp.zeros((ROWS_PER_TILE, D), jnp.float32)
    partials = _sc_aggregate(x, row, col, zeros_blk)
    return _tc_mlp(x, partials[0, :N_NODES], partials[1, :N_NODES],
                   W1.T, b1.reshape(1, D), W2.T, b2.reshape(1, D))


# speedup vs baseline: 1.9873x; 1.9609x over previous
import functools

import jax
import jax.numpy as jnp
from jax import lax
from jax.experimental import pallas as pl
from jax.experimental.pallas import tpu as pltpu
from jax.experimental.pallas import tpu_sc as plsc

N_NODES = 10000
N_EDGES = 320000
D = 128

NC = 2
NS = 16
NW = NC * NS

EDGE_BATCH = 80
EDGES_PER_TILE = N_EDGES // NW
N_BATCHES = EDGES_PER_TILE // EDGE_BATCH
N_PAD = 10240
ROWS_PER_TILE = N_PAD // NS


def _sc_aggregate(x, row, col, zeros_blk):
    mesh = plsc.VectorSubcoreMesh(core_axis_name="c", subcore_axis_name="s")

    @functools.partial(
        pl.kernel,
        mesh=mesh,
        out_type=jax.ShapeDtypeStruct((NC, N_PAD, D), jnp.float32),
        scratch_types=[
            pltpu.VMEM((N_BATCHES, EDGE_BATCH), jnp.int32),
            pltpu.VMEM((N_BATCHES, EDGE_BATCH), jnp.int32),
            pltpu.VMEM((EDGE_BATCH, D), jnp.float32),
            pltpu.VMEM_SHARED((N_PAD, D), jnp.float32),
            pltpu.SemaphoreType.DMA,
        ],
    )
    def k(x_hbm, row_hbm, col_hbm, zeros_hbm, out_hbm,
          row_v, col_v, rows_v, agg, sem):
        c = lax.axis_index("c")
        s = lax.axis_index("s")
        wid = s * NC + c

        pltpu.sync_copy(zeros_hbm, agg.at[pl.ds(s * ROWS_PER_TILE, ROWS_PER_TILE)])
        pltpu.sync_copy(row_hbm.at[wid], row_v)
        pltpu.sync_copy(col_hbm.at[wid], col_v)
        plsc.subcore_barrier()

        def body(b, carry):
            pltpu.async_copy(x_hbm.at[row_v.at[b]], rows_v, sem).wait()
            pltpu.sync_copy(rows_v, agg.at[col_v.at[b]], add=True)
            return carry

        lax.fori_loop(0, N_BATCHES, body, 0)
        plsc.subcore_barrier()

        r0 = s * ROWS_PER_TILE
        pltpu.sync_copy(agg.at[pl.ds(r0, ROWS_PER_TILE)],
                        out_hbm.at[c, pl.ds(r0, ROWS_PER_TILE)])

    return k(x, row, col, zeros_blk)


def _mlp_body(x_ref, p0_ref, p1_ref, w1_ref, b1_ref, w2_ref, b2_ref, o_ref):
    h = x_ref[...] + p0_ref[...] + p1_ref[...]
    h1 = jnp.dot(h, w1_ref[...], preferred_element_type=jnp.float32) + b1_ref[...]
    h1 = jnp.maximum(h1, 0.0)
    o_ref[...] = jnp.dot(h1, w2_ref[...], preferred_element_type=jnp.float32) + b2_ref[...]


def _tc_mlp(x, p0, p1, w1t, b1, w2t, b2):
    block = 2000
    grid = (N_NODES // block,)
    row_spec = pl.BlockSpec((block, D), lambda i: (i, 0))
    full_spec = pl.BlockSpec((D, D), lambda i: (0, 0))
    bias_spec = pl.BlockSpec((1, D), lambda i: (0, 0))
    return pl.pallas_call(
        _mlp_body,
        grid=grid,
        in_specs=[row_spec, row_spec, row_spec, full_spec, bias_spec, full_spec, bias_spec],
        out_specs=row_spec,
        out_shape=jax.ShapeDtypeStruct((N_NODES, D), jnp.float32),
    )(x, p0, p1, w1t, b1, w2t, b2)


@jax.jit
def kernel(x, edge_index, W1, b1, W2, b2):
    row = edge_index[0].astype(jnp.int32).reshape(NW, N_BATCHES, EDGE_BATCH)
    col = edge_index[1].astype(jnp.int32).reshape(NW, N_BATCHES, EDGE_BATCH)
    zeros_blk = jnp.zeros((ROWS_PER_TILE, D), jnp.float32)
    partials = _sc_aggregate(x, row, col, zeros_blk)
    return _tc_mlp(x, partials[0, :N_NODES], partials[1, :N_NODES],
                   W1.T, b1.reshape(1, D), W2.T, b2.reshape(1, D))
